# baseline (speedup 1.0000x reference)
import math

import jax
import jax.numpy as jnp
from jax.experimental import pallas as pl

_BLOCK_ROWS = 2000


def _norm_body(x_ref, m_ref, inv_ref, o_ref):
    o_ref[...] = (x_ref[...] - m_ref[...]) * inv_ref[...]


def kernel(descriptors, mean, std):
    n, d = descriptors.shape
    inv = 1.0 / (std * math.sqrt(d))
    grid = pl.cdiv(n, _BLOCK_ROWS)
    return pl.pallas_call(
        _norm_body,
        grid=(grid,),
        in_specs=[
            pl.BlockSpec((_BLOCK_ROWS, d), lambda i: (i, 0)),
            pl.BlockSpec((1, d), lambda i: (0, 0)),
            pl.BlockSpec((1, d), lambda i: (0, 0)),
        ],
        out_specs=pl.BlockSpec((_BLOCK_ROWS, d), lambda i: (i, 0)),
        out_shape=jax.ShapeDtypeStruct((n, d), descriptors.dtype),
    )(descriptors, mean, inv)


# speedup vs baseline: 1.0146x; 1.0146x over previous
import math

import jax
import jax.numpy as jnp
from jax.experimental import pallas as pl

_BLOCK_ROWS = 4000


def _norm_body(x_ref, a_ref, b_ref, o_ref):
    o_ref[...] = x_ref[...] * a_ref[...] + b_ref[...]


def kernel(descriptors, mean, std):
    n, d = descriptors.shape
    a = 1.0 / (std * math.sqrt(d))
    b = -mean * a
    grid = pl.cdiv(n, _BLOCK_ROWS)
    return pl.pallas_call(
        _norm_body,
        grid=(grid,),
        in_specs=[
            pl.BlockSpec((_BLOCK_ROWS, d), lambda i: (i, 0)),
            pl.BlockSpec((1, d), lambda i: (0, 0)),
            pl.BlockSpec((1, d), lambda i: (0, 0)),
        ],
        out_specs=pl.BlockSpec((_BLOCK_ROWS, d), lambda i: (i, 0)),
        out_shape=jax.ShapeDtypeStruct((n, d), descriptors.dtype),
    )(descriptors, a, b)


# speedup vs baseline: 1.0193x; 1.0046x over previous
import math

import jax
import jax.numpy as jnp
from jax.experimental import pallas as pl

_BLOCK_ROWS = 5000


def _norm_body(x_ref, a_ref, b_ref, o_ref):
    o_ref[...] = x_ref[...] * a_ref[...] + b_ref[...]


def kernel(descriptors, mean, std):
    n, d = descriptors.shape
    a = 1.0 / (std * math.sqrt(d))
    b = -mean * a
    grid = pl.cdiv(n, _BLOCK_ROWS)
    return pl.pallas_call(
        _norm_body,
        grid=(grid,),
        in_specs=[
            pl.BlockSpec((_BLOCK_ROWS, d), lambda i: (i, 0)),
            pl.BlockSpec((1, d), lambda i: (0, 0)),
            pl.BlockSpec((1, d), lambda i: (0, 0)),
        ],
        out_specs=pl.BlockSpec((_BLOCK_ROWS, d), lambda i: (i, 0)),
        out_shape=jax.ShapeDtypeStruct((n, d), descriptors.dtype),
    )(descriptors, a, b)
